# Initial kernel scaffold; baseline (speedup 1.0000x reference)
import functools

import jax
import jax.numpy as jnp
from jax import lax
from jax.experimental import pallas as pl
from jax.experimental.pallas import tpu as pltpu
from jax.experimental.pallas import tpu_sc as plsc

N_NODES = 10000
N_EDGES = 320000
D_FEAT = 128

_NC = 2
_NS = 16
_NW = _NC * _NS
_E_PER_W = N_EDGES // _NW
_LANES = 16


def _matvec_body(x_ref, w_ref, b_ref, o_ref):
    o_ref[...] = lax.dot_general(
        w_ref[...], x_ref[...],
        (((1,), (1,)), ((), ())),
        preferred_element_type=jnp.float32,
    ) + b_ref[...]


def _node_tables(x, W, b):
    w_pad = jnp.zeros((8, D_FEAT), jnp.float32)
    w_pad = w_pad.at[0, :].set(W[0, :D_FEAT])
    w_pad = w_pad.at[1, :].set(W[0, D_FEAT:])
    b_pad = jnp.zeros((8, 1), jnp.float32).at[0, 0].set(b[0])
    return pl.pallas_call(
        _matvec_body,
        out_shape=jax.ShapeDtypeStruct((8, N_NODES), jnp.float32),
    )(x, w_pad, b_pad)


def _make_sc_kernel():
    mesh = plsc.VectorSubcoreMesh(core_axis_name="c", subcore_axis_name="s")

    @functools.partial(
        pl.kernel,
        mesh=mesh,
        out_type=jax.ShapeDtypeStruct((N_EDGES,), jnp.float32),
        scratch_types=[
            pltpu.VMEM((N_NODES,), jnp.float32),
            pltpu.VMEM((N_NODES,), jnp.float32),
            pltpu.VMEM((_E_PER_W,), jnp.int32),
            pltpu.VMEM((_E_PER_W,), jnp.int32),
            pltpu.VMEM((_E_PER_W,), jnp.float32),
        ],
    )
    def sc_edge_kernel(p_hbm, q_hbm, src_hbm, trg_hbm, out_hbm,
                       p_v, q_v, src_v, trg_v, out_v):
        wid = lax.axis_index("s") * _NC + lax.axis_index("c")
        base = wid * _E_PER_W
        pltpu.sync_copy(p_hbm, p_v)
        pltpu.sync_copy(q_hbm, q_v)
        pltpu.sync_copy(src_hbm.at[pl.ds(base, _E_PER_W)], src_v)
        pltpu.sync_copy(trg_hbm.at[pl.ds(base, _E_PER_W)], trg_v)

        def body(i, carry):
            off = i * _LANES
            si = src_v[pl.ds(off, _LANES)]
            ti = trg_v[pl.ds(off, _LANES)]
            pv = plsc.load_gather(p_v, [si])
            qv = plsc.load_gather(q_v, [ti])
            z = pv + qv
            out_v[pl.ds(off, _LANES)] = 1.0 / (1.0 + jnp.exp(-z))
            return carry

        lax.fori_loop(0, _E_PER_W // _LANES, body, 0)
        pltpu.sync_copy(out_v, out_hbm.at[pl.ds(base, _E_PER_W)])

    return sc_edge_kernel


_SC_KERNEL = _make_sc_kernel()


def kernel(input, edge_src_nodes, edge_trg_nodes, W, b):
    x = input.reshape(-1, input.shape[-1]).astype(jnp.float32)
    tables = _node_tables(x, W.astype(jnp.float32), b.astype(jnp.float32))
    p = tables[0]
    q = tables[1]
    src = edge_src_nodes.astype(jnp.int32)
    trg = edge_trg_nodes.astype(jnp.int32)
    out = _SC_KERNEL(p, q, src, trg)
    return out.reshape(N_EDGES, 1)


# speedup vs baseline: 30.5091x; 30.5091x over previous
import functools

import jax
import jax.numpy as jnp
from jax import lax
from jax.experimental import pallas as pl
from jax.experimental.pallas import tpu as pltpu
from jax.experimental.pallas import tpu_sc as plsc

N_NODES = 10000
N_EDGES = 320000
D_FEAT = 128

_NC = 2
_NS = 16
_NW = _NC * _NS
_E_PER_W = N_EDGES // _NW
_LANES = 16


def _matvec_body(x_ref, w_ref, b_ref, o_ref):
    o_ref[...] = lax.dot_general(
        w_ref[...], x_ref[...],
        (((1,), (1,)), ((), ())),
        preferred_element_type=jnp.float32,
    ) + b_ref[...]


def _node_tables(x, W, b):
    w_pad = jnp.zeros((8, D_FEAT), jnp.float32)
    w_pad = w_pad.at[0, :].set(W[0, :D_FEAT])
    w_pad = w_pad.at[1, :].set(W[0, D_FEAT:])
    b_pad = jnp.zeros((8, 1), jnp.float32).at[0, 0].set(b[0])
    return pl.pallas_call(
        _matvec_body,
        out_shape=jax.ShapeDtypeStruct((8, N_NODES), jnp.float32),
    )(x, w_pad, b_pad)


def _make_sc_kernel():
    mesh = plsc.VectorSubcoreMesh(core_axis_name="c", subcore_axis_name="s")

    @functools.partial(
        pl.kernel,
        mesh=mesh,
        out_type=jax.ShapeDtypeStruct((N_EDGES,), jnp.float32),
        compiler_params=pltpu.CompilerParams(needs_layout_passes=False),
        scratch_types=[
            pltpu.VMEM((N_NODES,), jnp.float32),
            pltpu.VMEM((N_NODES,), jnp.float32),
            pltpu.VMEM((_E_PER_W,), jnp.int32),
            pltpu.VMEM((_E_PER_W,), jnp.int32),
            pltpu.VMEM((_E_PER_W,), jnp.float32),
        ],
    )
    def sc_edge_kernel(p_hbm, q_hbm, src_hbm, trg_hbm, out_hbm,
                       p_v, q_v, src_v, trg_v, out_v):
        wid = lax.axis_index("s") * _NC + lax.axis_index("c")
        base = wid * _E_PER_W
        pltpu.sync_copy(p_hbm, p_v)
        pltpu.sync_copy(q_hbm, q_v)
        pltpu.sync_copy(src_hbm.at[pl.ds(base, _E_PER_W)], src_v)
        pltpu.sync_copy(trg_hbm.at[pl.ds(base, _E_PER_W)], trg_v)

        def body(i, carry):
            off = i * _LANES
            si = src_v[pl.ds(off, _LANES)]
            ti = trg_v[pl.ds(off, _LANES)]
            pv = plsc.load_gather(p_v, [si])
            qv = plsc.load_gather(q_v, [ti])
            z = pv + qv
            out_v[pl.ds(off, _LANES)] = 1.0 / (1.0 + jnp.exp(-z))
            return carry

        lax.fori_loop(0, _E_PER_W // _LANES, body, 0)
        pltpu.sync_copy(out_v, out_hbm.at[pl.ds(base, _E_PER_W)])

    return sc_edge_kernel


_SC_KERNEL = _make_sc_kernel()


def kernel(input, edge_src_nodes, edge_trg_nodes, W, b):
    x = input.reshape(-1, input.shape[-1]).astype(jnp.float32)
    tables = _node_tables(x, W.astype(jnp.float32), b.astype(jnp.float32))
    p = tables[0]
    q = tables[1]
    src = edge_src_nodes.astype(jnp.int32)
    trg = edge_trg_nodes.astype(jnp.int32)
    out = _SC_KERNEL(p, q, src, trg)
    return out.reshape(N_EDGES, 1)


# speedup vs baseline: 45.5822x; 1.4940x over previous
import functools

import jax
import jax.numpy as jnp
from jax import lax
from jax.experimental import pallas as pl
from jax.experimental.pallas import tpu as pltpu
from jax.experimental.pallas import tpu_sc as plsc

N_NODES = 10000
N_EDGES = 320000
D_FEAT = 128

_NC = 2
_NS = 16
_NW = _NC * _NS
_E_PER_W = N_EDGES // _NW
_LANES = 16
_UNROLL = 8


def _matvec_body(x_ref, w_ref, b_ref, p_ref, q_ref):
    out = lax.dot_general(
        w_ref[...], x_ref[...],
        (((1,), (1,)), ((), ())),
        preferred_element_type=jnp.float32,
    )
    p_ref[...] = out[0] + b_ref[0]
    q_ref[...] = out[1]


def _node_tables(x, W, b):
    w2 = W.reshape(2, D_FEAT)
    return pl.pallas_call(
        _matvec_body,
        in_specs=[
            pl.BlockSpec(memory_space=pltpu.VMEM),
            pl.BlockSpec(memory_space=pltpu.VMEM),
            pl.BlockSpec(memory_space=pltpu.SMEM),
        ],
        out_shape=(
            jax.ShapeDtypeStruct((N_NODES,), jnp.float32),
            jax.ShapeDtypeStruct((N_NODES,), jnp.float32),
        ),
    )(x, w2, b)


def _make_sc_kernel():
    mesh = plsc.VectorSubcoreMesh(core_axis_name="c", subcore_axis_name="s")

    @functools.partial(
        pl.kernel,
        mesh=mesh,
        out_type=jax.ShapeDtypeStruct((N_EDGES,), jnp.float32),
        compiler_params=pltpu.CompilerParams(needs_layout_passes=False),
        scratch_types=[
            pltpu.VMEM((N_NODES,), jnp.float32),
            pltpu.VMEM((N_NODES,), jnp.float32),
            pltpu.VMEM((_E_PER_W,), jnp.int32),
            pltpu.VMEM((_E_PER_W,), jnp.int32),
            pltpu.VMEM((_E_PER_W,), jnp.float32),
            pltpu.SemaphoreType.DMA,
        ],
    )
    def sc_edge_kernel(p_hbm, q_hbm, src_hbm, trg_hbm, out_hbm,
                       p_v, q_v, src_v, trg_v, out_v, sem):
        wid = lax.axis_index("s") * _NC + lax.axis_index("c")
        base = wid * _E_PER_W
        c1 = pltpu.async_copy(p_hbm, p_v, sem)
        c2 = pltpu.async_copy(q_hbm, q_v, sem)
        c3 = pltpu.async_copy(src_hbm.at[pl.ds(base, _E_PER_W)], src_v, sem)
        c4 = pltpu.async_copy(trg_hbm.at[pl.ds(base, _E_PER_W)], trg_v, sem)
        c1.wait()
        c2.wait()
        c3.wait()
        c4.wait()

        @plsc.parallel_loop(0, _E_PER_W // _LANES, 1, unroll=_UNROLL)
        def _body(i):
            off = i * _LANES
            si = src_v[pl.ds(off, _LANES)]
            ti = trg_v[pl.ds(off, _LANES)]
            pv = plsc.load_gather(p_v, [si])
            qv = plsc.load_gather(q_v, [ti])
            z = pv + qv
            out_v[pl.ds(off, _LANES)] = 1.0 / (1.0 + jnp.exp(-z))

        pltpu.sync_copy(out_v, out_hbm.at[pl.ds(base, _E_PER_W)])

    return sc_edge_kernel


_SC_KERNEL = _make_sc_kernel()


def kernel(input, edge_src_nodes, edge_trg_nodes, W, b):
    x = input.reshape(-1, input.shape[-1]).astype(jnp.float32)
    p, q = _node_tables(x, W.astype(jnp.float32), b.astype(jnp.float32))
    src = edge_src_nodes.astype(jnp.int32)
    trg = edge_trg_nodes.astype(jnp.int32)
    return _SC_KERNEL(p, q, src, trg).reshape(N_EDGES, 1)
